# baseline (speedup 1.0000x reference)
import jax
import jax.numpy as jnp
from jax.experimental import pallas as pl
from jax.experimental.pallas import tpu as pltpu

N_NODES = 10000
HEADS = 8
HIDDEN = 16
NUM_CLASSES = 16


def _log_softmax_body(x_ref, o_ref):
    x = x_ref[...]
    m = jnp.max(x, axis=-1, keepdims=True)
    ex = jnp.exp(x - m)
    s = jnp.sum(ex, axis=-1, keepdims=True)
    o_ref[...] = (x - m) - jnp.log(s)


def _log_softmax(x):
    return pl.pallas_call(
        _log_softmax_body,
        out_shape=jax.ShapeDtypeStruct(x.shape, x.dtype),
    )(x)


def _gat_conv(x, src, dst, W, a_src, a_dst, b, heads, out_ch, concat):
    N = x.shape[0]
    xp = (x @ W).reshape(N, heads, out_ch)
    alpha_src = jnp.sum(xp * a_src[None, :, :], axis=-1)
    alpha_dst = jnp.sum(xp * a_dst[None, :, :], axis=-1)
    e = alpha_src[src] + alpha_dst[dst]
    e = jax.nn.leaky_relu(e, negative_slope=0.2)
    m = jax.ops.segment_max(e, dst, num_segments=N)
    m = jnp.where(jnp.isfinite(m), m, 0.0)
    ex = jnp.exp(e - m[dst])
    s = jax.ops.segment_sum(ex, dst, num_segments=N)
    alpha = ex / (s[dst] + 1e-16)
    msg = alpha[:, :, None] * xp[src]
    out = jax.ops.segment_sum(msg, dst, num_segments=N)
    if concat:
        out = out.reshape(N, heads * out_ch)
    else:
        out = out.mean(axis=1)
    return out + b


def kernel(x, edge_index, W1, att_src1, att_dst1, bias1, W2, att_src2, att_dst2, bias2):
    N = x.shape[0]
    ar = jnp.arange(N, dtype=edge_index.dtype)
    ei = jnp.concatenate([edge_index, jnp.stack([ar, ar], axis=0)], axis=1)
    src = ei[0]
    dst = ei[1]
    h = _gat_conv(x, src, dst, W1, att_src1, att_dst1, bias1, HEADS, HIDDEN, True)
    h = jax.nn.elu(h)
    h = _gat_conv(h, src, dst, W2, att_src2, att_dst2, bias2, 1, NUM_CLASSES, False)
    return _log_softmax(h)


# speedup vs baseline: 33.3662x; 33.3662x over previous
import functools

import jax
import jax.numpy as jnp
from jax import lax
from jax.experimental import pallas as pl
from jax.experimental.pallas import tpu as pltpu
from jax.experimental.pallas import tpu_sc as plsc

N_NODES = 10000
N_EDGES = 320000
D_FEAT = 128
HEADS = 8
HIDDEN = 16
NUM_CLASSES = 16

N_PAD = 10112
ROWS_PER_TILE = N_PAD // 16
TRASH = N_NODES
CHUNK = 128
N_WORKERS = 32
E_PAD = 323584
E_PER_W = E_PAD // N_WORKERS
N_CHUNKS = E_PER_W // CHUNK


def _leaky(z):
    return jnp.maximum(z, 0.2 * z)


def _tc1_body(x_ref, w1_ref, as_ref, ad_ref, p_ref, xp_ref, asrcp_ref, dpack_ref):
    xp = jnp.dot(x_ref[...], w1_ref[...], preferred_element_type=jnp.float32)
    s = jnp.dot(xp, as_ref[...], preferred_element_type=jnp.float32)
    d = jnp.dot(xp, ad_ref[...], preferred_element_type=jnp.float32)
    m = _leaky(s + d)
    dpack = d + jnp.dot(m, p_ref[...], preferred_element_type=jnp.float32)
    xp_ref[...] = xp
    asrcp_ref[...] = s
    dpack_ref[...] = dpack


def _tc1(x, W1, As, Ad, P):
    n = x.shape[0]
    blk = 1000
    grid = n // blk
    return pl.pallas_call(
        _tc1_body,
        grid=(grid,),
        in_specs=[
            pl.BlockSpec((blk, D_FEAT), lambda i: (i, 0)),
            pl.BlockSpec((D_FEAT, D_FEAT), lambda i: (0, 0)),
            pl.BlockSpec((D_FEAT, 16), lambda i: (0, 0)),
            pl.BlockSpec((D_FEAT, 16), lambda i: (0, 0)),
            pl.BlockSpec((16, 16), lambda i: (0, 0)),
        ],
        out_specs=[
            pl.BlockSpec((blk, D_FEAT), lambda i: (i, 0)),
            pl.BlockSpec((blk, 16), lambda i: (i, 0)),
            pl.BlockSpec((blk, 16), lambda i: (i, 0)),
        ],
        out_shape=[
            jax.ShapeDtypeStruct((n, D_FEAT), jnp.float32),
            jax.ShapeDtypeStruct((n, 16), jnp.float32),
            jax.ShapeDtypeStruct((n, 16), jnp.float32),
        ],
    )(x, W1, As, Ad, P)


def _sc1_body(asrcp_hbm, dpack_hbm, xp_hbm, sidx_hbm, didx_hbm, z128_hbm, z16_hbm,
              acc_out, ssum_out,
              acc_sh, ssum_sh, sidx_v, didx_v, sbuf, dbuf, xbuf, exbuf, msgbuf,
              sem1, sem2, sem3):
    c = lax.axis_index("c")
    s = lax.axis_index("s")
    w = c * 16 + s
    r0 = s * ROWS_PER_TILE
    pltpu.sync_copy(z128_hbm, acc_sh.at[pl.ds(r0, ROWS_PER_TILE)])
    pltpu.sync_copy(z16_hbm, ssum_sh.at[pl.ds(r0, ROWS_PER_TILE)])
    plsc.subcore_barrier()

    base0 = w * E_PER_W

    def chunk_body(i, carry):
        base = base0 + i * CHUNK
        pltpu.sync_copy(sidx_hbm.at[pl.ds(base, CHUNK)], sidx_v)
        pltpu.sync_copy(didx_hbm.at[pl.ds(base, CHUNK)], didx_v)
        g1 = pltpu.async_copy(asrcp_hbm.at[sidx_v], sbuf, sem1)
        g2 = pltpu.async_copy(dpack_hbm.at[didx_v], dbuf, sem2)
        g3 = pltpu.async_copy(xp_hbm.at[sidx_v], xbuf, sem3)
        g1.wait()
        g2.wait()
        g3.wait()

        def edge_body(k, carry2):
            sv = sbuf[k, :]
            dv = dbuf[k, :]
            u = jnp.exp(_leaky(sv + dv) - jnp.flip(dv, 0))
            exbuf[k, :] = u
            for h in range(HEADS):
                exh = plsc.load_gather(
                    exbuf, [jnp.full((16,), k, jnp.int32),
                            jnp.full((16,), h, jnp.int32)])
                xv = xbuf[k, pl.ds(h * 16, 16)]
                msgbuf[k, pl.ds(h * 16, 16)] = xv * exh
            return carry2

        lax.fori_loop(0, CHUNK, edge_body, 0, unroll=2)
        pltpu.sync_copy(exbuf, ssum_sh.at[didx_v], add=True)
        pltpu.sync_copy(msgbuf, acc_sh.at[didx_v], add=True)
        return carry

    lax.fori_loop(0, N_CHUNKS, chunk_body, 0)
    plsc.subcore_barrier()
    pltpu.sync_copy(acc_sh.at[pl.ds(r0, ROWS_PER_TILE)],
                    acc_out.at[c, pl.ds(r0, ROWS_PER_TILE)])
    pltpu.sync_copy(ssum_sh.at[pl.ds(r0, ROWS_PER_TILE)],
                    ssum_out.at[c, pl.ds(r0, ROWS_PER_TILE)])


def _sc1(asrcp, dpack, xp, sidx, didx, z128, z16):
    mesh = plsc.VectorSubcoreMesh(core_axis_name="c", subcore_axis_name="s")
    f = pl.kernel(
        _sc1_body,
        out_type=[
            jax.ShapeDtypeStruct((2, N_PAD, D_FEAT), jnp.float32),
            jax.ShapeDtypeStruct((2, N_PAD, 16), jnp.float32),
        ],
        mesh=mesh,
        scratch_types=[
            pltpu.VMEM_SHARED((N_PAD, D_FEAT), jnp.float32),
            pltpu.VMEM_SHARED((N_PAD, 16), jnp.float32),
            pltpu.VMEM((CHUNK,), jnp.int32),
            pltpu.VMEM((CHUNK,), jnp.int32),
            pltpu.VMEM((CHUNK, 16), jnp.float32),
            pltpu.VMEM((CHUNK, 16), jnp.float32),
            pltpu.VMEM((CHUNK, D_FEAT), jnp.float32),
            pltpu.VMEM((CHUNK, 16), jnp.float32),
            pltpu.VMEM((CHUNK, D_FEAT), jnp.float32),
            pltpu.SemaphoreType.DMA,
            pltpu.SemaphoreType.DMA,
            pltpu.SemaphoreType.DMA,
        ],
        compiler_params=pltpu.CompilerParams(needs_layout_passes=False, use_tc_tiling_on_sc=False),
    )
    return f(asrcp, dpack, xp, sidx, didx, z128, z16)


def _tc2_body(acc0_ref, acc1_ref, ss0_ref, ss1_ref, xp_ref, b1_ref, w2_ref,
              r_ref, s2m_ref, d2m_ref,
              srow2_ref, drow2_ref, xp2_ref):
    stot = 1.0 + ss0_ref[...] + ss1_ref[...]
    srep = jnp.dot(stot, r_ref[...], preferred_element_type=jnp.float32)
    acc = xp_ref[...] + acc0_ref[...] + acc1_ref[...]
    h1 = acc / (srep + 1e-16) + b1_ref[...]
    h1 = jnp.where(h1 > 0, h1, jnp.exp(h1) - 1.0)
    xp2 = jnp.dot(h1, w2_ref[...], preferred_element_type=jnp.float32)
    s2 = jnp.dot(xp2, s2m_ref[...], preferred_element_type=jnp.float32)
    d2 = jnp.dot(xp2, d2m_ref[...], preferred_element_type=jnp.float32)
    m2 = _leaky(s2 + d2)
    col = lax.broadcasted_iota(jnp.int32, d2.shape, 1)
    drow2_ref[...] = jnp.where(col < 8, d2, m2)
    srow2_ref[...] = jnp.concatenate([xp2, s2], axis=1)
    xp2_ref[...] = xp2


def _tc2(acc0, acc1, ss0, ss1, xp, b1, W2, R, S2m, D2m):
    n = xp.shape[0]
    blk = 1000
    grid = n // blk
    return pl.pallas_call(
        _tc2_body,
        grid=(grid,),
        in_specs=[
            pl.BlockSpec((blk, D_FEAT), lambda i: (i, 0)),
            pl.BlockSpec((blk, D_FEAT), lambda i: (i, 0)),
            pl.BlockSpec((blk, 16), lambda i: (i, 0)),
            pl.BlockSpec((blk, 16), lambda i: (i, 0)),
            pl.BlockSpec((blk, D_FEAT), lambda i: (i, 0)),
            pl.BlockSpec((1, D_FEAT), lambda i: (0, 0)),
            pl.BlockSpec((D_FEAT, 16), lambda i: (0, 0)),
            pl.BlockSpec((16, D_FEAT), lambda i: (0, 0)),
            pl.BlockSpec((16, 16), lambda i: (0, 0)),
            pl.BlockSpec((16, 16), lambda i: (0, 0)),
        ],
        out_specs=[
            pl.BlockSpec((blk, 32), lambda i: (i, 0)),
            pl.BlockSpec((blk, 16), lambda i: (i, 0)),
            pl.BlockSpec((blk, 16), lambda i: (i, 0)),
        ],
        out_shape=[
            jax.ShapeDtypeStruct((n, 32), jnp.float32),
            jax.ShapeDtypeStruct((n, 16), jnp.float32),
            jax.ShapeDtypeStruct((n, 16), jnp.float32),
        ],
    )(acc0, acc1, ss0, ss1, xp, b1, W2, R, S2m, D2m)


def _sc2_body(srow2_hbm, drow2_hbm, sidx_hbm, didx_hbm, z16_hbm,
              acc_out, ssum_out,
              acc_sh, ssum_sh, sidx_v, didx_v, sbuf, dbuf, exbuf, msgbuf,
              sem1, sem2):
    c = lax.axis_index("c")
    s = lax.axis_index("s")
    w = c * 16 + s
    r0 = s * ROWS_PER_TILE
    pltpu.sync_copy(z16_hbm, acc_sh.at[pl.ds(r0, ROWS_PER_TILE)])
    pltpu.sync_copy(z16_hbm, ssum_sh.at[pl.ds(r0, ROWS_PER_TILE)])
    plsc.subcore_barrier()

    base0 = w * E_PER_W

    def chunk_body(i, carry):
        base = base0 + i * CHUNK
        pltpu.sync_copy(sidx_hbm.at[pl.ds(base, CHUNK)], sidx_v)
        pltpu.sync_copy(didx_hbm.at[pl.ds(base, CHUNK)], didx_v)
        g1 = pltpu.async_copy(srow2_hbm.at[sidx_v], sbuf, sem1)
        g2 = pltpu.async_copy(drow2_hbm.at[didx_v], dbuf, sem2)
        g1.wait()
        g2.wait()

        def edge_body(k, carry2):
            xv = sbuf[k, pl.ds(0, 16)]
            av = sbuf[k, pl.ds(16, 16)]
            dv = dbuf[k, :]
            u = jnp.exp(_leaky(av + dv) - jnp.flip(dv, 0))
            exbuf[k, :] = u
            exs = plsc.load_gather(
                exbuf, [jnp.full((16,), k, jnp.int32),
                        jnp.zeros((16,), jnp.int32)])
            msgbuf[k, :] = xv * exs
            return carry2

        lax.fori_loop(0, CHUNK, edge_body, 0, unroll=4)
        pltpu.sync_copy(exbuf, ssum_sh.at[didx_v], add=True)
        pltpu.sync_copy(msgbuf, acc_sh.at[didx_v], add=True)
        return carry

    lax.fori_loop(0, N_CHUNKS, chunk_body, 0)
    plsc.subcore_barrier()
    pltpu.sync_copy(acc_sh.at[pl.ds(r0, ROWS_PER_TILE)],
                    acc_out.at[c, pl.ds(r0, ROWS_PER_TILE)])
    pltpu.sync_copy(ssum_sh.at[pl.ds(r0, ROWS_PER_TILE)],
                    ssum_out.at[c, pl.ds(r0, ROWS_PER_TILE)])


def _sc2(srow2, drow2, sidx, didx, z16):
    mesh = plsc.VectorSubcoreMesh(core_axis_name="c", subcore_axis_name="s")
    f = pl.kernel(
        _sc2_body,
        out_type=[
            jax.ShapeDtypeStruct((2, N_PAD, 16), jnp.float32),
            jax.ShapeDtypeStruct((2, N_PAD, 16), jnp.float32),
        ],
        mesh=mesh,
        scratch_types=[
            pltpu.VMEM_SHARED((N_PAD, 16), jnp.float32),
            pltpu.VMEM_SHARED((N_PAD, 16), jnp.float32),
            pltpu.VMEM((CHUNK,), jnp.int32),
            pltpu.VMEM((CHUNK,), jnp.int32),
            pltpu.VMEM((CHUNK, 32), jnp.float32),
            pltpu.VMEM((CHUNK, 16), jnp.float32),
            pltpu.VMEM((CHUNK, 16), jnp.float32),
            pltpu.VMEM((CHUNK, 16), jnp.float32),
            pltpu.SemaphoreType.DMA,
            pltpu.SemaphoreType.DMA,
        ],
        compiler_params=pltpu.CompilerParams(needs_layout_passes=False, use_tc_tiling_on_sc=False),
    )
    return f(srow2, drow2, sidx, didx, z16)


def _tc3_body(acc0_ref, acc1_ref, ss0_ref, ss1_ref, xp2_ref, b2_ref, c_ref,
              out_ref):
    ssum = jnp.dot(ss0_ref[...] + ss1_ref[...], c_ref[...],
                   preferred_element_type=jnp.float32)
    logits = (xp2_ref[...] + acc0_ref[...] + acc1_ref[...]) / (1.0 + ssum + 1e-16)
    logits = logits + b2_ref[...]
    m = jnp.max(logits, axis=-1, keepdims=True)
    ex = jnp.exp(logits - m)
    ssf = jnp.sum(ex, axis=-1, keepdims=True)
    out_ref[...] = (logits - m) - jnp.log(ssf)


def _tc3(acc0, acc1, ss0, ss1, xp2, b2, C):
    n = xp2.shape[0]
    blk = 1000
    grid = n // blk
    return pl.pallas_call(
        _tc3_body,
        grid=(grid,),
        in_specs=[
            pl.BlockSpec((blk, 16), lambda i: (i, 0)),
            pl.BlockSpec((blk, 16), lambda i: (i, 0)),
            pl.BlockSpec((blk, 16), lambda i: (i, 0)),
            pl.BlockSpec((blk, 16), lambda i: (i, 0)),
            pl.BlockSpec((blk, 16), lambda i: (i, 0)),
            pl.BlockSpec((1, 16), lambda i: (0, 0)),
            pl.BlockSpec((16, 16), lambda i: (0, 0)),
        ],
        out_specs=pl.BlockSpec((blk, 16), lambda i: (i, 0)),
        out_shape=jax.ShapeDtypeStruct((n, 16), jnp.float32),
    )(acc0, acc1, ss0, ss1, xp2, b2, C)


def kernel(x, edge_index, W1, att_src1, att_dst1, bias1, W2, att_src2,
           att_dst2, bias2):
    f32 = jnp.float32
    i32 = jnp.int32

    eye8 = jnp.eye(8, 16, dtype=f32)
    As = (att_src1[:, :, None] * eye8[:, None, :]).reshape(D_FEAT, 16)
    Ad = (att_dst1[:, :, None] * eye8[:, None, :]).reshape(D_FEAT, 16)
    j = jnp.arange(16)
    P = jnp.where((j[:, None] < 8) & (j[None, :] == 15 - j[:, None]), 1.0,
                  0.0).astype(f32)
    R = jnp.where(j[:, None] == (jnp.arange(D_FEAT)[None, :] // 16), 1.0,
                  0.0).astype(f32) * jnp.where(j[:, None] < 8, 1.0, 0.0)
    S2m = jnp.broadcast_to(att_src2.reshape(16, 1), (16, 16)).astype(f32)
    D2m = jnp.broadcast_to(att_dst2.reshape(16, 1), (16, 16)).astype(f32)
    C = jnp.broadcast_to(j[:, None] == 0, (16, 16)).astype(f32)

    sidx = jnp.concatenate(
        [edge_index[0].astype(i32), jnp.zeros((E_PAD - N_EDGES,), i32)])
    didx = jnp.concatenate(
        [edge_index[1].astype(i32),
         jnp.full((E_PAD - N_EDGES,), TRASH, i32)])

    z128 = jnp.zeros((ROWS_PER_TILE, D_FEAT), f32)
    z16 = jnp.zeros((ROWS_PER_TILE, 16), f32)

    xp, asrcp, dpack = _tc1(x, W1, As, Ad, P)
    pad = [(0, N_PAD - N_NODES), (0, 0)]
    xp_p = jnp.pad(xp, pad)
    asrcp_p = jnp.pad(asrcp, pad)
    dpack_p = jnp.pad(dpack, pad)

    acc, ssum = _sc1(asrcp_p, dpack_p, xp_p, sidx, didx, z128, z16)

    srow2, drow2, xp2 = _tc2(acc[0, :N_NODES], acc[1, :N_NODES],
                             ssum[0, :N_NODES], ssum[1, :N_NODES],
                             xp, bias1.reshape(1, D_FEAT), W2, R, S2m, D2m)
    srow2_p = jnp.pad(srow2, pad)
    drow2_p = jnp.pad(drow2, pad)

    acc2, ssum2 = _sc2(srow2_p, drow2_p, sidx, didx, z16)

    return _tc3(acc2[0, :N_NODES], acc2[1, :N_NODES],
                ssum2[0, :N_NODES], ssum2[1, :N_NODES],
                xp2, bias2.reshape(1, 16), C)


# speedup vs baseline: 85.2912x; 2.5562x over previous
import jax
import jax.numpy as jnp
from jax import lax
from jax.experimental import pallas as pl
from jax.experimental.pallas import tpu as pltpu
from jax.experimental.pallas import tpu_sc as plsc

N_NODES = 10000
N_EDGES = 320000
D_FEAT = 128
HEADS = 8
HIDDEN = 16
NUM_CLASSES = 16

N_PAD = 10112
ROWS_PER_TILE = N_PAD // 16
TRASH = N_NODES
N_WORKERS = 32
CHUNK1 = 56
E_PAD1 = 322560
E_PER_W1 = E_PAD1 // N_WORKERS
N_CHUNKS1 = E_PER_W1 // CHUNK1
CHUNK2 = 128
E_PAD2 = 327680
E_PER_W2 = E_PAD2 // N_WORKERS
N_CHUNKS2 = E_PER_W2 // CHUNK2

XS_W = D_FEAT + 16
AC_W = D_FEAT + 16


def _leaky(z):
    return jnp.maximum(z, 0.2 * z)


def _tc1_body(x_ref, w1_ref, as_ref, ad_ref, p_ref, xs_ref, dpack_ref):
    xp = jnp.dot(x_ref[...], w1_ref[...], preferred_element_type=jnp.float32)
    s = jnp.dot(xp, as_ref[...], preferred_element_type=jnp.float32)
    d = jnp.dot(xp, ad_ref[...], preferred_element_type=jnp.float32)
    m = _leaky(s + d)
    dpack_ref[...] = d + jnp.dot(m, p_ref[...], preferred_element_type=jnp.float32)
    xs_ref[...] = jnp.concatenate([xp, s], axis=1)


def _tc1(x_p, W1, As, Ad, P):
    blk = ROWS_PER_TILE
    grid = N_PAD // blk
    return pl.pallas_call(
        _tc1_body,
        grid=(grid,),
        in_specs=[
            pl.BlockSpec((blk, D_FEAT), lambda i: (i, 0)),
            pl.BlockSpec((D_FEAT, D_FEAT), lambda i: (0, 0)),
            pl.BlockSpec((D_FEAT, 16), lambda i: (0, 0)),
            pl.BlockSpec((D_FEAT, 16), lambda i: (0, 0)),
            pl.BlockSpec((16, 16), lambda i: (0, 0)),
        ],
        out_specs=[
            pl.BlockSpec((blk, XS_W), lambda i: (i, 0)),
            pl.BlockSpec((blk, 16), lambda i: (i, 0)),
        ],
        out_shape=[
            jax.ShapeDtypeStruct((N_PAD, XS_W), jnp.float32),
            jax.ShapeDtypeStruct((N_PAD, 16), jnp.float32),
        ],
    )(x_p, W1, As, Ad, P)


def _sc1_body(xs_hbm, dpack_hbm, sidx_hbm, didx_hbm, z144_hbm,
              accs_out,
              accs_sh,
              sidx0, didx0, sidx1, didx1, sidx2, didx2, sidx3, didx3,
              xsbuf0, dbuf0, mbuf0,
              xsbuf1, dbuf1, mbuf1,
              semg0, sems0, semg1, sems1):
    c = lax.axis_index("c")
    s = lax.axis_index("s")
    w = c * 16 + s
    r0 = s * ROWS_PER_TILE
    sidx = (sidx0, sidx1, sidx2, sidx3)
    didx = (didx0, didx1, didx2, didx3)
    xsbuf = (xsbuf0, xsbuf1)
    dbuf = (dbuf0, dbuf1)
    mbuf = (mbuf0, mbuf1)
    semg = (semg0, semg1)
    sems = (sems0, sems1)

    pltpu.sync_copy(z144_hbm, accs_sh.at[pl.ds(r0, ROWS_PER_TILE)])
    plsc.subcore_barrier()

    base0 = w * E_PER_W1

    def fire_gathers(g, b, i4):
        base = base0 + g * CHUNK1
        pltpu.sync_copy(sidx_hbm.at[pl.ds(base, CHUNK1)], sidx[i4])
        pltpu.sync_copy(didx_hbm.at[pl.ds(base, CHUNK1)], didx[i4])
        pltpu.async_copy(xs_hbm.at[sidx[i4]], xsbuf[b], semg[b])
        pltpu.async_copy(dpack_hbm.at[didx[i4]], dbuf[b], semg[b])

    def drain_gathers(b, i4):
        pltpu.make_async_copy(xs_hbm.at[sidx[i4]], xsbuf[b], semg[b]).wait()
        pltpu.make_async_copy(dpack_hbm.at[didx[i4]], dbuf[b], semg[b]).wait()

    def fire_scatters(b, i4):
        pltpu.async_copy(mbuf[b], accs_sh.at[didx[i4]], sems[b], add=True)

    def drain_scatters(b, i4):
        pltpu.make_async_copy(mbuf[b], accs_sh.at[didx[i4]], sems[b]).wait()

    fire_gathers(0, 0, 0)
    fire_gathers(1, 1, 1)

    def quad_body(g4, carry):
        for q in range(4):
            g = g4 * 4 + q
            b = q % 2

            @pl.when(g >= 2)
            def _():
                drain_scatters(b, (q + 2) % 4)

            drain_gathers(b, q)

            @plsc.parallel_loop(0, CHUNK1, unroll=4)
            def edge_body(k):
                sv = xsbuf[b][k, pl.ds(D_FEAT, 16)]
                dv = dbuf[b][k, :]
                u = jnp.exp(_leaky(sv + dv) - jnp.flip(dv, 0))
                mbuf[b][k, pl.ds(D_FEAT, 16)] = u
                for h in range(HEADS):
                    exh = jnp.broadcast_to(u[h], (16,))
                    xv = xsbuf[b][k, pl.ds(h * 16, 16)]
                    mbuf[b][k, pl.ds(h * 16, 16)] = xv * exh

            fire_scatters(b, q)

            @pl.when(g + 2 < N_CHUNKS1)
            def _():
                fire_gathers(g + 2, b, (q + 2) % 4)
        return carry

    lax.fori_loop(0, N_CHUNKS1 // 4, quad_body, 0)
    drain_scatters(0, 2)
    drain_scatters(1, 3)
    plsc.subcore_barrier()
    pltpu.sync_copy(accs_sh.at[pl.ds(r0, ROWS_PER_TILE)],
                    accs_out.at[c, pl.ds(r0, ROWS_PER_TILE)])


def _sc1(xs, dpack, sidx, didx, z144):
    mesh = plsc.VectorSubcoreMesh(core_axis_name="c", subcore_axis_name="s")
    idx_buf = [pltpu.VMEM((CHUNK1,), jnp.int32)] * 8
    buf_set = [
        pltpu.VMEM((CHUNK1, XS_W), jnp.float32),
        pltpu.VMEM((CHUNK1, 16), jnp.float32),
        pltpu.VMEM((CHUNK1, AC_W), jnp.float32),
    ]
    f = pl.kernel(
        _sc1_body,
        out_type=[
            jax.ShapeDtypeStruct((2, N_PAD, AC_W), jnp.float32),
        ],
        mesh=mesh,
        scratch_types=[
            pltpu.VMEM_SHARED((N_PAD, AC_W), jnp.float32),
            *idx_buf,
            *buf_set,
            *buf_set,
            pltpu.SemaphoreType.DMA,
            pltpu.SemaphoreType.DMA,
            pltpu.SemaphoreType.DMA,
            pltpu.SemaphoreType.DMA,
        ],
        compiler_params=pltpu.CompilerParams(needs_layout_passes=False, use_tc_tiling_on_sc=False),
    )
    return f(xs, dpack, sidx, didx, z144)


def _tc2_body(a0_ref, a1_ref, xs_ref, b1_ref, w2_ref, r_ref, s2m_ref, d2m_ref,
              srow2_ref, drow2_ref, xp2_ref):
    a0 = a0_ref[0]
    a1 = a1_ref[0]
    stot = 1.0 + a0[:, D_FEAT:] + a1[:, D_FEAT:]
    srep = jnp.dot(stot, r_ref[...], preferred_element_type=jnp.float32)
    acc = xs_ref[:, :D_FEAT] + a0[:, :D_FEAT] + a1[:, :D_FEAT]
    h1 = acc / (srep + 1e-16) + b1_ref[...]
    h1 = jnp.where(h1 > 0, h1, jnp.exp(h1) - 1.0)
    xp2 = jnp.dot(h1, w2_ref[...], preferred_element_type=jnp.float32)
    s2 = jnp.dot(xp2, s2m_ref[...], preferred_element_type=jnp.float32)
    d2 = jnp.dot(xp2, d2m_ref[...], preferred_element_type=jnp.float32)
    m2 = _leaky(s2 + d2)
    col = lax.broadcasted_iota(jnp.int32, d2.shape, 1)
    drow2_ref[...] = jnp.where(col < 8, d2, m2)
    srow2_ref[...] = jnp.concatenate([xp2, s2], axis=1)
    xp2_ref[...] = xp2


def _tc2(accs, xs, b1, W2, R, S2m, D2m):
    blk = ROWS_PER_TILE
    grid = N_PAD // blk
    return pl.pallas_call(
        _tc2_body,
        grid=(grid,),
        in_specs=[
            pl.BlockSpec((1, blk, AC_W), lambda i: (0, i, 0)),
            pl.BlockSpec((1, blk, AC_W), lambda i: (1, i, 0)),
            pl.BlockSpec((blk, XS_W), lambda i: (i, 0)),
            pl.BlockSpec((1, D_FEAT), lambda i: (0, 0)),
            pl.BlockSpec((D_FEAT, 16), lambda i: (0, 0)),
            pl.BlockSpec((16, D_FEAT), lambda i: (0, 0)),
            pl.BlockSpec((16, 16), lambda i: (0, 0)),
            pl.BlockSpec((16, 16), lambda i: (0, 0)),
        ],
        out_specs=[
            pl.BlockSpec((blk, 32), lambda i: (i, 0)),
            pl.BlockSpec((blk, 16), lambda i: (i, 0)),
            pl.BlockSpec((blk, 16), lambda i: (i, 0)),
        ],
        out_shape=[
            jax.ShapeDtypeStruct((N_PAD, 32), jnp.float32),
            jax.ShapeDtypeStruct((N_PAD, 16), jnp.float32),
            jax.ShapeDtypeStruct((N_PAD, 16), jnp.float32),
        ],
    )(accs, accs, xs, b1, W2, R, S2m, D2m)


def _sc2_body(srow2_hbm, drow2_hbm, sidx_hbm, didx_hbm, z32_hbm,
              accs_out,
              accs_sh,
              sidx0, didx0, sidx1, didx1, sidx2, didx2, sidx3, didx3,
              sbuf0, dbuf0, mbuf0,
              sbuf1, dbuf1, mbuf1,
              semg0, sems0, semg1, sems1):
    c = lax.axis_index("c")
    s = lax.axis_index("s")
    w = c * 16 + s
    r0 = s * ROWS_PER_TILE
    sidx = (sidx0, sidx1, sidx2, sidx3)
    didx = (didx0, didx1, didx2, didx3)
    sbuf = (sbuf0, sbuf1)
    dbuf = (dbuf0, dbuf1)
    mbuf = (mbuf0, mbuf1)
    semg = (semg0, semg1)
    sems = (sems0, sems1)

    pltpu.sync_copy(z32_hbm, accs_sh.at[pl.ds(r0, ROWS_PER_TILE)])
    plsc.subcore_barrier()

    base0 = w * E_PER_W2

    def fire_gathers(g, b, i4):
        base = base0 + g * CHUNK2
        pltpu.sync_copy(sidx_hbm.at[pl.ds(base, CHUNK2)], sidx[i4])
        pltpu.sync_copy(didx_hbm.at[pl.ds(base, CHUNK2)], didx[i4])
        pltpu.async_copy(srow2_hbm.at[sidx[i4]], sbuf[b], semg[b])
        pltpu.async_copy(drow2_hbm.at[didx[i4]], dbuf[b], semg[b])

    def drain_gathers(b, i4):
        pltpu.make_async_copy(srow2_hbm.at[sidx[i4]], sbuf[b], semg[b]).wait()
        pltpu.make_async_copy(drow2_hbm.at[didx[i4]], dbuf[b], semg[b]).wait()

    def fire_scatters(b, i4):
        pltpu.async_copy(mbuf[b], accs_sh.at[didx[i4]], sems[b], add=True)

    def drain_scatters(b, i4):
        pltpu.make_async_copy(mbuf[b], accs_sh.at[didx[i4]], sems[b]).wait()

    fire_gathers(0, 0, 0)
    fire_gathers(1, 1, 1)

    def quad_body(g4, carry):
        for q in range(4):
            g = g4 * 4 + q
            b = q % 2

            @pl.when(g >= 2)
            def _():
                drain_scatters(b, (q + 2) % 4)

            drain_gathers(b, q)

            @plsc.parallel_loop(0, CHUNK2, unroll=8)
            def edge_body(k):
                xv = sbuf[b][k, pl.ds(0, 16)]
                av = sbuf[b][k, pl.ds(16, 16)]
                dv = dbuf[b][k, :]
                u = jnp.exp(_leaky(av + dv) - jnp.flip(dv, 0))
                mbuf[b][k, pl.ds(16, 16)] = u
                mbuf[b][k, pl.ds(0, 16)] = xv * jnp.broadcast_to(u[0], (16,))

            fire_scatters(b, q)

            @pl.when(g + 2 < N_CHUNKS2)
            def _():
                fire_gathers(g + 2, b, (q + 2) % 4)
        return carry

    lax.fori_loop(0, N_CHUNKS2 // 4, quad_body, 0)
    drain_scatters(0, 2)
    drain_scatters(1, 3)
    plsc.subcore_barrier()
    pltpu.sync_copy(accs_sh.at[pl.ds(r0, ROWS_PER_TILE)],
                    accs_out.at[c, pl.ds(r0, ROWS_PER_TILE)])


def _sc2(srow2, drow2, sidx, didx, z32):
    mesh = plsc.VectorSubcoreMesh(core_axis_name="c", subcore_axis_name="s")
    idx_buf = [pltpu.VMEM((CHUNK2,), jnp.int32)] * 8
    buf_set = [
        pltpu.VMEM((CHUNK2, 32), jnp.float32),
        pltpu.VMEM((CHUNK2, 16), jnp.float32),
        pltpu.VMEM((CHUNK2, 32), jnp.float32),
    ]
    f = pl.kernel(
        _sc2_body,
        out_type=[
            jax.ShapeDtypeStruct((2, N_PAD, 32), jnp.float32),
        ],
        mesh=mesh,
        scratch_types=[
            pltpu.VMEM_SHARED((N_PAD, 32), jnp.float32),
            *idx_buf,
            *buf_set,
            *buf_set,
            pltpu.SemaphoreType.DMA,
            pltpu.SemaphoreType.DMA,
            pltpu.SemaphoreType.DMA,
            pltpu.SemaphoreType.DMA,
        ],
        compiler_params=pltpu.CompilerParams(needs_layout_passes=False, use_tc_tiling_on_sc=False),
    )
    return f(srow2, drow2, sidx, didx, z32)


def _tc3_body(a0_ref, a1_ref, xp2_ref, b2_ref, c_ref, out_ref):
    a0 = a0_ref[0]
    a1 = a1_ref[0]
    ssum = jnp.dot(a0[:, 16:] + a1[:, 16:], c_ref[...],
                   preferred_element_type=jnp.float32)
    logits = (xp2_ref[...] + a0[:, :16] + a1[:, :16]) / (1.0 + ssum + 1e-16)
    logits = logits + b2_ref[...]
    m = jnp.max(logits, axis=-1, keepdims=True)
    ex = jnp.exp(logits - m)
    ssf = jnp.sum(ex, axis=-1, keepdims=True)
    out_ref[...] = (logits - m) - jnp.log(ssf)


def _tc3(accs2, xp2, b2, C):
    blk = ROWS_PER_TILE
    grid = N_PAD // blk
    return pl.pallas_call(
        _tc3_body,
        grid=(grid,),
        in_specs=[
            pl.BlockSpec((1, blk, 32), lambda i: (0, i, 0)),
            pl.BlockSpec((1, blk, 32), lambda i: (1, i, 0)),
            pl.BlockSpec((blk, 16), lambda i: (i, 0)),
            pl.BlockSpec((1, 16), lambda i: (0, 0)),
            pl.BlockSpec((16, 16), lambda i: (0, 0)),
        ],
        out_specs=pl.BlockSpec((blk, 16), lambda i: (i, 0)),
        out_shape=jax.ShapeDtypeStruct((N_PAD, 16), jnp.float32),
    )(accs2, accs2, xp2, b2, C)


def kernel(x, edge_index, W1, att_src1, att_dst1, bias1, W2, att_src2,
           att_dst2, bias2):
    f32 = jnp.float32
    i32 = jnp.int32

    eye8 = jnp.eye(8, 16, dtype=f32)
    As = (att_src1[:, :, None] * eye8[:, None, :]).reshape(D_FEAT, 16)
    Ad = (att_dst1[:, :, None] * eye8[:, None, :]).reshape(D_FEAT, 16)
    j = jnp.arange(16)
    P = jnp.where((j[:, None] < 8) & (j[None, :] == 15 - j[:, None]), 1.0,
                  0.0).astype(f32)
    R = jnp.where(j[:, None] == (jnp.arange(D_FEAT)[None, :] // 16), 1.0,
                  0.0).astype(f32) * jnp.where(j[:, None] < 8, 1.0, 0.0)
    S2m = jnp.broadcast_to(att_src2.reshape(16, 1), (16, 16)).astype(f32)
    D2m = jnp.broadcast_to(att_dst2.reshape(16, 1), (16, 16)).astype(f32)
    C = jnp.broadcast_to(j[:, None] == 0, (16, 16)).astype(f32)

    sidx = jnp.concatenate(
        [edge_index[0].astype(i32), jnp.zeros((E_PAD2 - N_EDGES,), i32)])
    didx = jnp.concatenate(
        [edge_index[1].astype(i32),
         TRASH + jnp.arange(E_PAD2 - N_EDGES, dtype=i32) % (N_PAD - N_NODES)])

    z144 = jnp.zeros((ROWS_PER_TILE, AC_W), f32)
    z32 = jnp.zeros((ROWS_PER_TILE, 32), f32)
    x_p = jnp.pad(x, [(0, N_PAD - N_NODES), (0, 0)])

    xs, dpack = _tc1(x_p, W1, As, Ad, P)

    (accs,) = _sc1(xs, dpack, sidx, didx, z144)

    srow2, drow2, xp2 = _tc2(accs, xs, bias1.reshape(1, D_FEAT), W2, R, S2m,
                             D2m)

    (accs2,) = _sc2(srow2, drow2, sidx, didx, z32)

    return _tc3(accs2, xp2, bias2.reshape(1, 16), C)[:N_NODES]
